# baseline (speedup 1.0000x reference)
import functools

import jax
import jax.numpy as jnp
from jax import lax
from jax.experimental import pallas as pl
from jax.experimental.pallas import tpu as pltpu
from jax.experimental.pallas import tpu_sc as plsc

N = 50000
H = 128
NQ = 4
QW = 12512
ACCR = QW + 32
RPSQ = ACCR // 16
ZROWS = 16
NZ = 49
E = 500000
NW = 32
K = 256
B = 64
EPW = K * B
KC = 64
RB = 1000
GRID = N // RB
RBA = 3128
GRIDA = 16


def _pad_src(ix):
    pad = NW * EPW - E
    ix = jnp.concatenate([ix.astype(jnp.int32), jnp.zeros((pad,), jnp.int32)])
    return ix.reshape(NW, K, B)


def _quarter_dst(ix):
    pad = NW * EPW - E
    ix = jnp.concatenate([ix.astype(jnp.int32), jnp.full((pad,), N, jnp.int32)])
    out = []
    for q in range(NQ):
        lo = q * QW
        rel = ix - lo
        out.append(jnp.where((rel >= 0) & (rel < QW), rel,
                             QW).reshape(NW, K, B))
    return out


def _encode_body(x_ref, w_ref, b_ref, o_ref):
    o_ref[...] = jax.nn.relu(
        jnp.dot(x_ref[...], w_ref[...], preferred_element_type=jnp.float32)
        + b_ref[...])


def _encode(x, w, b):
    return pl.pallas_call(
        _encode_body,
        grid=(GRID,),
        in_specs=[
            pl.BlockSpec((RB, 64), lambda i: (i, 0)),
            pl.BlockSpec((64, H), lambda i: (0, 0)),
            pl.BlockSpec((1, H), lambda i: (0, 0)),
        ],
        out_specs=pl.BlockSpec((RB, H), lambda i: (i, 0)),
        out_shape=jax.ShapeDtypeStruct((N, H), jnp.float32),
    )(x, w, b.reshape(1, H))


def _pre_body(h_ref, w_ref, o_ref):
    o_ref[...] = jnp.dot(h_ref[...], w_ref[...],
                         preferred_element_type=jnp.float32)


def _pretransform(h, w):
    return pl.pallas_call(
        _pre_body,
        grid=(GRID,),
        in_specs=[
            pl.BlockSpec((RB, H), lambda i: (i, 0)),
            pl.BlockSpec((H, H), lambda i: (0, 0)),
        ],
        out_specs=pl.BlockSpec((RB, H), lambda i: (i, 0)),
        out_shape=jax.ShapeDtypeStruct((N, H), jnp.float32),
    )(h, w)


def _asm_body(decode, p_ref, h_ref, wr_ref, br_ref, *rest):
    p = p_ref[...]
    agg = p[0, 0] + p[1, 0]
    h = h_ref[...]
    out = agg + br_ref[...] + jnp.dot(h, wr_ref[...],
                                      preferred_element_type=jnp.float32)
    new = jax.nn.relu(out) + h
    if decode:
        dw_ref, db_ref, o_ref, y_ref = rest
        o_ref[...] = new
        y_ref[...] = jnp.dot(new, dw_ref[...],
                             preferred_element_type=jnp.float32) + db_ref[...]
    else:
        (o_ref,) = rest
        o_ref[...] = new


def _assemble(partials, h, w_root, b_rel, dec_w=None, dec_b=None):
    decode = dec_w is not None
    in_specs = [
        pl.BlockSpec((2, 1, RBA, H), lambda i: (0, i // 4, i % 4, 0)),
        pl.BlockSpec((RBA, H), lambda i: (i, 0)),
        pl.BlockSpec((H, H), lambda i: (0, 0)),
        pl.BlockSpec((1, H), lambda i: (0, 0)),
    ]
    args = [partials, h, w_root, b_rel.reshape(1, H)]
    out_specs = [pl.BlockSpec((RBA, H), lambda i: (i, 0))]
    out_shape = [jax.ShapeDtypeStruct((N, H), jnp.float32)]
    if decode:
        in_specs += [pl.BlockSpec((H, 1), lambda i: (0, 0)),
                     pl.BlockSpec((1, 1), lambda i: (0, 0))]
        args += [dec_w, dec_b.reshape(1, 1)]
        out_specs.append(pl.BlockSpec((RBA, 1), lambda i: (i, 0)))
        out_shape.append(jax.ShapeDtypeStruct((N, 1), jnp.float32))
    return pl.pallas_call(
        functools.partial(_asm_body, decode),
        grid=(GRIDA,),
        in_specs=in_specs,
        out_specs=out_specs,
        out_shape=out_shape,
    )(*args)


def _make_sc_accum(num_types):
    nt = num_types
    mesh = plsc.VectorSubcoreMesh(core_axis_name="c", subcore_axis_name="s")

    @functools.partial(
        pl.kernel,
        out_type=jax.ShapeDtypeStruct((2, NQ, ACCR, H), jnp.float32),
        mesh=mesh,
        compiler_params=pltpu.CompilerParams(use_tc_tiling_on_sc=False),
        scratch_types=[
            pltpu.VMEM_SHARED((ACCR, H), jnp.float32),
            pltpu.VMEM((KC, B), jnp.int32),
            pltpu.VMEM((KC, B), jnp.int32),
            pltpu.VMEM((B, H), jnp.float32),
            pltpu.VMEM((B, H), jnp.float32),
            pltpu.VMEM((ZROWS, H), jnp.float32),
            pltpu.SemaphoreType.DMA,
            pltpu.SemaphoreType.DMA,
        ],
    )
    def body(*refs):
        g_refs = refs[:nt]
        sidx_h = refs[nt:2 * nt]
        didx_h = refs[2 * nt:2 * nt + nt * NQ]
        p_ref = refs[2 * nt + nt * NQ]
        acc, sidx, didx, rows0, rows1, zbuf, sem0, sem1 = \
            refs[2 * nt + nt * NQ + 1:]

        ci = lax.axis_index("c")
        si = lax.axis_index("s")
        wid = si * 2 + ci

        z16 = jnp.zeros((16,), jnp.float32)

        def zfill(r, carry):
            for qq in range(H // 16):
                zbuf[r, pl.ds(qq * 16, 16)] = z16
            return carry

        lax.fori_loop(0, ZROWS, zfill, 0)

        for q in range(NQ):
            def zero(z, carry):
                pltpu.sync_copy(zbuf, acc.at[pl.ds(si * RPSQ + z * ZROWS,
                                                   ZROWS)])
                return carry

            lax.fori_loop(0, NZ, zero, 0)
            plsc.subcore_barrier()
            for t in range(nt):
                g = g_refs[t]

                def chunk(m, carry, t=t, q=q, g=g):
                    pltpu.sync_copy(sidx_h[t].at[wid, pl.ds(m * KC, KC)],
                                    sidx)
                    pltpu.sync_copy(didx_h[t * NQ + q].at[wid,
                                                          pl.ds(m * KC, KC)],
                                    didx)
                    pltpu.async_copy(g.at[sidx.at[0]], rows0, sem0)

                    def step(i, c2):
                        j0 = 2 * i
                        pltpu.async_copy(g.at[sidx.at[j0 + 1]], rows1, sem1)
                        pltpu.make_async_copy(g.at[sidx.at[j0]], rows0,
                                              sem0).wait()
                        pltpu.sync_copy(rows0, acc.at[didx.at[j0]], add=True)

                        @pl.when(j0 + 2 < KC)
                        def _():
                            pltpu.async_copy(g.at[sidx.at[j0 + 2]], rows0,
                                             sem0)

                        pltpu.make_async_copy(g.at[sidx.at[j0 + 1]], rows1,
                                              sem1).wait()
                        pltpu.sync_copy(rows1, acc.at[didx.at[j0 + 1]],
                                        add=True)
                        return c2

                    lax.fori_loop(0, KC // 2, step, 0)
                    return carry

                lax.fori_loop(0, K // KC, chunk, 0)
            plsc.subcore_barrier()
            pltpu.sync_copy(acc.at[pl.ds(si * RPSQ, RPSQ)],
                            p_ref.at[ci, q, pl.ds(si * RPSQ, RPSQ)])
            plsc.subcore_barrier()

    return body


_sc_accum_1 = _make_sc_accum(1)
_sc_accum_3 = _make_sc_accum(3)


def kernel(x_base, x_joint, x_foot, edge_index_base_gt_base,
           edge_index_base_gs_base, edge_index_base_to_joint,
           edge_index_joint_to_joint, edge_index_joint_to_foot,
           edge_index_foot_to_joint, params):
    idx = {}
    for name, ei in (("b2j", edge_index_base_to_joint),
                     ("j2j", edge_index_joint_to_joint),
                     ("j2f", edge_index_joint_to_foot),
                     ("f2j", edge_index_foot_to_joint)):
        idx[name] = (_pad_src(ei[0]), _quarter_dst(ei[1]))

    h0 = {
        "base": _encode(x_base, params["enc_base_W"], params["enc_base_b"]),
        "joint": _encode(x_joint, params["enc_joint_W"], params["enc_joint_b"]),
        "foot": _encode(x_foot, params["enc_foot_W"], params["enc_foot_b"]),
    }

    p0 = {e: params["l0_%s" % n] for e, n in (
        ("b2j", "base_to_joint"), ("j2j", "joint_to_joint"),
        ("f2j", "foot_to_joint"), ("j2f", "joint_to_foot"))}
    p1_j2f = params["l1_joint_to_foot"]

    g_b2j = _pretransform(h0["base"], p0["b2j"]["Wrel"])
    g_j2j = _pretransform(h0["joint"], p0["j2j"]["Wrel"])
    g_f2j = _pretransform(h0["foot"], p0["f2j"]["Wrel"])
    part_j = _sc_accum_3(g_b2j, g_j2j, g_f2j,
                         idx["b2j"][0], idx["j2j"][0], idx["f2j"][0],
                         *idx["b2j"][1], *idx["j2j"][1], *idx["f2j"][1])
    g_j2f = _pretransform(h0["joint"], p0["j2f"]["Wrel"])
    part_f = _sc_accum_1(g_j2f, idx["j2f"][0], *idx["j2f"][1])

    w_root_j = p0["b2j"]["Wroot"] + p0["j2j"]["Wroot"] + p0["f2j"]["Wroot"]
    b_rel_j = p0["b2j"]["brel"] + p0["j2j"]["brel"] + p0["f2j"]["brel"]
    (h1_joint,) = _assemble(part_j, h0["joint"], w_root_j, b_rel_j)
    (h1_foot,) = _assemble(part_f, h0["foot"], p0["j2f"]["Wroot"],
                           p0["j2f"]["brel"])

    g1_j2f = _pretransform(h1_joint, p1_j2f["Wrel"])
    part_f1 = _sc_accum_1(g1_j2f, idx["j2f"][0], *idx["j2f"][1])
    _, y = _assemble(part_f1, h1_foot, p1_j2f["Wroot"], p1_j2f["brel"],
                     dec_w=params["dec_W"], dec_b=params["dec_b"])
    return y


# speedup vs baseline: 1.0065x; 1.0065x over previous
import functools

import jax
import jax.numpy as jnp
from jax import lax
from jax.experimental import pallas as pl
from jax.experimental.pallas import tpu as pltpu
from jax.experimental.pallas import tpu_sc as plsc

N = 50000
H = 128
NQ = 4
QW = 12512
ACCR = QW + 32
RPSQ = ACCR // 16
ZROWS = 16
NZ = 49
E = 500000
NW = 32
K = 256
B = 64
EPW = K * B
KC = 64
RB = 1000
GRID = N // RB
RBA = 3128
GRIDA = 16


def _pad_src(ix):
    pad = NW * EPW - E
    ix = jnp.concatenate([ix.astype(jnp.int32), jnp.zeros((pad,), jnp.int32)])
    return ix.reshape(NW, K, B)


def _quarter_dst(ix):
    pad = NW * EPW - E
    ix = jnp.concatenate([ix.astype(jnp.int32), jnp.full((pad,), N, jnp.int32)])
    dump = QW + (jnp.arange(NW * EPW, dtype=jnp.int32) & 31)
    out = []
    for q in range(NQ):
        lo = q * QW
        rel = ix - lo
        out.append(jnp.where((rel >= 0) & (rel < QW), rel,
                             dump).reshape(NW, K, B))
    return out


def _encode_body(x_ref, w_ref, b_ref, o_ref):
    o_ref[...] = jax.nn.relu(
        jnp.dot(x_ref[...], w_ref[...], preferred_element_type=jnp.float32)
        + b_ref[...])


def _encode(x, w, b):
    return pl.pallas_call(
        _encode_body,
        grid=(GRID,),
        in_specs=[
            pl.BlockSpec((RB, 64), lambda i: (i, 0)),
            pl.BlockSpec((64, H), lambda i: (0, 0)),
            pl.BlockSpec((1, H), lambda i: (0, 0)),
        ],
        out_specs=pl.BlockSpec((RB, H), lambda i: (i, 0)),
        out_shape=jax.ShapeDtypeStruct((N, H), jnp.float32),
    )(x, w, b.reshape(1, H))


def _pre_body(h_ref, w_ref, o_ref):
    o_ref[...] = jnp.dot(h_ref[...], w_ref[...],
                         preferred_element_type=jnp.float32)


def _pretransform(h, w):
    return pl.pallas_call(
        _pre_body,
        grid=(GRID,),
        in_specs=[
            pl.BlockSpec((RB, H), lambda i: (i, 0)),
            pl.BlockSpec((H, H), lambda i: (0, 0)),
        ],
        out_specs=pl.BlockSpec((RB, H), lambda i: (i, 0)),
        out_shape=jax.ShapeDtypeStruct((N, H), jnp.float32),
    )(h, w)


def _asm_body(decode, p_ref, h_ref, wr_ref, br_ref, *rest):
    p = p_ref[...]
    agg = p[0, 0] + p[1, 0]
    h = h_ref[...]
    out = agg + br_ref[...] + jnp.dot(h, wr_ref[...],
                                      preferred_element_type=jnp.float32)
    new = jax.nn.relu(out) + h
    if decode:
        dw_ref, db_ref, o_ref, y_ref = rest
        o_ref[...] = new
        y_ref[...] = jnp.dot(new, dw_ref[...],
                             preferred_element_type=jnp.float32) + db_ref[...]
    else:
        (o_ref,) = rest
        o_ref[...] = new


def _assemble(partials, h, w_root, b_rel, dec_w=None, dec_b=None):
    decode = dec_w is not None
    in_specs = [
        pl.BlockSpec((2, 1, RBA, H), lambda i: (0, i // 4, i % 4, 0)),
        pl.BlockSpec((RBA, H), lambda i: (i, 0)),
        pl.BlockSpec((H, H), lambda i: (0, 0)),
        pl.BlockSpec((1, H), lambda i: (0, 0)),
    ]
    args = [partials, h, w_root, b_rel.reshape(1, H)]
    out_specs = [pl.BlockSpec((RBA, H), lambda i: (i, 0))]
    out_shape = [jax.ShapeDtypeStruct((N, H), jnp.float32)]
    if decode:
        in_specs += [pl.BlockSpec((H, 1), lambda i: (0, 0)),
                     pl.BlockSpec((1, 1), lambda i: (0, 0))]
        args += [dec_w, dec_b.reshape(1, 1)]
        out_specs.append(pl.BlockSpec((RBA, 1), lambda i: (i, 0)))
        out_shape.append(jax.ShapeDtypeStruct((N, 1), jnp.float32))
    return pl.pallas_call(
        functools.partial(_asm_body, decode),
        grid=(GRIDA,),
        in_specs=in_specs,
        out_specs=out_specs,
        out_shape=out_shape,
    )(*args)


def _make_sc_accum(num_types):
    nt = num_types
    mesh = plsc.VectorSubcoreMesh(core_axis_name="c", subcore_axis_name="s")

    @functools.partial(
        pl.kernel,
        out_type=jax.ShapeDtypeStruct((2, NQ, ACCR, H), jnp.float32),
        mesh=mesh,
        compiler_params=pltpu.CompilerParams(use_tc_tiling_on_sc=False),
        scratch_types=[
            pltpu.VMEM_SHARED((ACCR, H), jnp.float32),
            pltpu.VMEM((KC, B), jnp.int32),
            pltpu.VMEM((KC, B), jnp.int32),
            pltpu.VMEM((B, H), jnp.float32),
            pltpu.VMEM((B, H), jnp.float32),
            pltpu.VMEM((ZROWS, H), jnp.float32),
            pltpu.SemaphoreType.DMA,
            pltpu.SemaphoreType.DMA,
        ],
    )
    def body(*refs):
        g_refs = refs[:nt]
        sidx_h = refs[nt:2 * nt]
        didx_h = refs[2 * nt:2 * nt + nt * NQ]
        p_ref = refs[2 * nt + nt * NQ]
        acc, sidx, didx, rows0, rows1, zbuf, sem0, sem1 = \
            refs[2 * nt + nt * NQ + 1:]

        ci = lax.axis_index("c")
        si = lax.axis_index("s")
        wid = si * 2 + ci

        z16 = jnp.zeros((16,), jnp.float32)

        def zfill(r, carry):
            for qq in range(H // 16):
                zbuf[r, pl.ds(qq * 16, 16)] = z16
            return carry

        lax.fori_loop(0, ZROWS, zfill, 0)

        for q in range(NQ):
            def zero(z, carry):
                pltpu.sync_copy(zbuf, acc.at[pl.ds(si * RPSQ + z * ZROWS,
                                                   ZROWS)])
                return carry

            lax.fori_loop(0, NZ, zero, 0)
            plsc.subcore_barrier()
            for t in range(nt):
                g = g_refs[t]

                def chunk(m, carry, t=t, q=q, g=g):
                    pltpu.sync_copy(sidx_h[t].at[wid, pl.ds(m * KC, KC)],
                                    sidx)
                    pltpu.sync_copy(didx_h[t * NQ + q].at[wid,
                                                          pl.ds(m * KC, KC)],
                                    didx)
                    pltpu.async_copy(g.at[sidx.at[0]], rows0, sem0)

                    def step(i, c2):
                        j0 = 2 * i
                        pltpu.async_copy(g.at[sidx.at[j0 + 1]], rows1, sem1)
                        pltpu.make_async_copy(g.at[sidx.at[j0]], rows0,
                                              sem0).wait()
                        pltpu.sync_copy(rows0, acc.at[didx.at[j0]], add=True)

                        @pl.when(j0 + 2 < KC)
                        def _():
                            pltpu.async_copy(g.at[sidx.at[j0 + 2]], rows0,
                                             sem0)

                        pltpu.make_async_copy(g.at[sidx.at[j0 + 1]], rows1,
                                              sem1).wait()
                        pltpu.sync_copy(rows1, acc.at[didx.at[j0 + 1]],
                                        add=True)
                        return c2

                    lax.fori_loop(0, KC // 2, step, 0)
                    return carry

                lax.fori_loop(0, K // KC, chunk, 0)
            plsc.subcore_barrier()
            pltpu.sync_copy(acc.at[pl.ds(si * RPSQ, RPSQ)],
                            p_ref.at[ci, q, pl.ds(si * RPSQ, RPSQ)])
            plsc.subcore_barrier()

    return body


_sc_accum_1 = _make_sc_accum(1)
_sc_accum_3 = _make_sc_accum(3)


def kernel(x_base, x_joint, x_foot, edge_index_base_gt_base,
           edge_index_base_gs_base, edge_index_base_to_joint,
           edge_index_joint_to_joint, edge_index_joint_to_foot,
           edge_index_foot_to_joint, params):
    idx = {}
    for name, ei in (("b2j", edge_index_base_to_joint),
                     ("j2j", edge_index_joint_to_joint),
                     ("j2f", edge_index_joint_to_foot),
                     ("f2j", edge_index_foot_to_joint)):
        idx[name] = (_pad_src(ei[0]), _quarter_dst(ei[1]))

    h0 = {
        "base": _encode(x_base, params["enc_base_W"], params["enc_base_b"]),
        "joint": _encode(x_joint, params["enc_joint_W"], params["enc_joint_b"]),
        "foot": _encode(x_foot, params["enc_foot_W"], params["enc_foot_b"]),
    }

    p0 = {e: params["l0_%s" % n] for e, n in (
        ("b2j", "base_to_joint"), ("j2j", "joint_to_joint"),
        ("f2j", "foot_to_joint"), ("j2f", "joint_to_foot"))}
    p1_j2f = params["l1_joint_to_foot"]

    g_b2j = _pretransform(h0["base"], p0["b2j"]["Wrel"])
    g_j2j = _pretransform(h0["joint"], p0["j2j"]["Wrel"])
    g_f2j = _pretransform(h0["foot"], p0["f2j"]["Wrel"])
    part_j = _sc_accum_3(g_b2j, g_j2j, g_f2j,
                         idx["b2j"][0], idx["j2j"][0], idx["f2j"][0],
                         *idx["b2j"][1], *idx["j2j"][1], *idx["f2j"][1])
    g_j2f = _pretransform(h0["joint"], p0["j2f"]["Wrel"])
    part_f = _sc_accum_1(g_j2f, idx["j2f"][0], *idx["j2f"][1])

    w_root_j = p0["b2j"]["Wroot"] + p0["j2j"]["Wroot"] + p0["f2j"]["Wroot"]
    b_rel_j = p0["b2j"]["brel"] + p0["j2j"]["brel"] + p0["f2j"]["brel"]
    (h1_joint,) = _assemble(part_j, h0["joint"], w_root_j, b_rel_j)
    (h1_foot,) = _assemble(part_f, h0["foot"], p0["j2f"]["Wroot"],
                           p0["j2f"]["brel"])

    g1_j2f = _pretransform(h1_joint, p1_j2f["Wrel"])
    part_f1 = _sc_accum_1(g1_j2f, idx["j2f"][0], *idx["j2f"][1])
    _, y = _assemble(part_f1, h1_foot, p1_j2f["Wroot"], p1_j2f["brel"],
                     dec_w=params["dec_W"], dec_b=params["dec_b"])
    return y


# speedup vs baseline: 1.8872x; 1.8750x over previous
import functools

import jax
import jax.numpy as jnp
from jax import lax
from jax.experimental import pallas as pl
from jax.experimental.pallas import tpu as pltpu
from jax.experimental.pallas import tpu_sc as plsc

N = 50000
H = 128
NQ = 4
QW = 12512
ACCR = QW + 32
RPSQ = ACCR // 16
ZROWS = 16
NZ = 49
E = 500000
NW = 32
K = 256
B = 64
EPW = K * B
KC = 64
CHE = 512
RPC = CHE // B
CAPC = EPW // CHE + 1
CAP = CAPC * CHE
STG = 2 * CHE + 16
RB = 1000
GRID = N // RB
RBA = 3128
GRIDA = 16


def _pad_src(ix):
    pad = NW * EPW - E
    ix = jnp.concatenate([ix.astype(jnp.int32), jnp.zeros((pad,), jnp.int32)])
    return ix.reshape(NW, K, B)


def _pad_dst(ix):
    pad = NW * EPW - E
    ix = jnp.concatenate([ix.astype(jnp.int32), jnp.full((pad,), N, jnp.int32)])
    return ix.reshape(NW, K, B)


def _encode_body(x_ref, w_ref, b_ref, o_ref):
    o_ref[...] = jax.nn.relu(
        jnp.dot(x_ref[...], w_ref[...], preferred_element_type=jnp.float32)
        + b_ref[...])


def _encode(x, w, b):
    return pl.pallas_call(
        _encode_body,
        grid=(GRID,),
        in_specs=[
            pl.BlockSpec((RB, 64), lambda i: (i, 0)),
            pl.BlockSpec((64, H), lambda i: (0, 0)),
            pl.BlockSpec((1, H), lambda i: (0, 0)),
        ],
        out_specs=pl.BlockSpec((RB, H), lambda i: (i, 0)),
        out_shape=jax.ShapeDtypeStruct((N, H), jnp.float32),
    )(x, w, b.reshape(1, H))


def _pre_body(h_ref, w_ref, o_ref):
    o_ref[...] = jnp.dot(h_ref[...], w_ref[...],
                         preferred_element_type=jnp.float32)


def _pretransform(h, w):
    return pl.pallas_call(
        _pre_body,
        grid=(GRID,),
        in_specs=[
            pl.BlockSpec((RB, H), lambda i: (i, 0)),
            pl.BlockSpec((H, H), lambda i: (0, 0)),
        ],
        out_specs=pl.BlockSpec((RB, H), lambda i: (i, 0)),
        out_shape=jax.ShapeDtypeStruct((N, H), jnp.float32),
    )(h, w)


def _asm_body(decode, p_ref, h_ref, wr_ref, br_ref, *rest):
    p = p_ref[...]
    agg = p[0, 0] + p[1, 0]
    h = h_ref[...]
    out = agg + br_ref[...] + jnp.dot(h, wr_ref[...],
                                      preferred_element_type=jnp.float32)
    new = jax.nn.relu(out) + h
    if decode:
        dw_ref, db_ref, o_ref, y_ref = rest
        o_ref[...] = new
        y_ref[...] = jnp.dot(new, dw_ref[...],
                             preferred_element_type=jnp.float32) + db_ref[...]
    else:
        (o_ref,) = rest
        o_ref[...] = new


def _assemble(partials, h, w_root, b_rel, dec_w=None, dec_b=None):
    decode = dec_w is not None
    in_specs = [
        pl.BlockSpec((2, 1, RBA, H), lambda i: (0, i // 4, i % 4, 0)),
        pl.BlockSpec((RBA, H), lambda i: (i, 0)),
        pl.BlockSpec((H, H), lambda i: (0, 0)),
        pl.BlockSpec((1, H), lambda i: (0, 0)),
    ]
    args = [partials, h, w_root, b_rel.reshape(1, H)]
    out_specs = [pl.BlockSpec((RBA, H), lambda i: (i, 0))]
    out_shape = [jax.ShapeDtypeStruct((N, H), jnp.float32)]
    if decode:
        in_specs += [pl.BlockSpec((H, 1), lambda i: (0, 0)),
                     pl.BlockSpec((1, 1), lambda i: (0, 0))]
        args += [dec_w, dec_b.reshape(1, 1)]
        out_specs.append(pl.BlockSpec((RBA, 1), lambda i: (i, 0)))
        out_shape.append(jax.ShapeDtypeStruct((N, 1), jnp.float32))
    return pl.pallas_call(
        functools.partial(_asm_body, decode),
        grid=(GRIDA,),
        in_specs=in_specs,
        out_specs=out_specs,
        out_shape=out_shape,
    )(*args)


_MESH = plsc.VectorSubcoreMesh(core_axis_name="c", subcore_axis_name="s")
_UNTILED = pltpu.CompilerParams(use_tc_tiling_on_sc=False,
                                needs_layout_passes=False)
_NSETS = 4


@functools.partial(
    pl.kernel,
    out_type=(
        [jax.ShapeDtypeStruct((NW, NQ, CAP), jnp.int32)
         for _ in range(2 * _NSETS)]
        + [jax.ShapeDtypeStruct((NW, 16), jnp.int32) for _ in range(_NSETS)]
    ),
    mesh=_MESH,
    compiler_params=_UNTILED,
    scratch_types=(
        [pltpu.VMEM((KC, B), jnp.int32),
         pltpu.VMEM((KC, B), jnp.int32)]
        + [pltpu.VMEM((STG,), jnp.int32) for _ in range(2 * NQ)]
        + [pltpu.VMEM((16,), jnp.int32)]
    ),
)
def _sc_bucket(*refs):
    src_h = refs[:_NSETS]
    dst_h = refs[_NSETS:2 * _NSETS]
    bs_h = refs[2 * _NSETS:3 * _NSETS]
    bd_h = refs[3 * _NSETS:4 * _NSETS]
    ns_h = refs[4 * _NSETS:5 * _NSETS]
    scr = refs[5 * _NSETS:]
    sidxc, didxc = scr[0], scr[1]
    stg_s = scr[2:2 + NQ]
    stg_d = scr[2 + NQ:2 + 2 * NQ]
    nsv = scr[2 + 2 * NQ]

    ci = lax.axis_index("c")
    si = lax.axis_index("s")
    wid = si * 2 + ci
    iota16 = lax.iota(jnp.int32, 16)
    zeros16 = jnp.zeros((16,), jnp.int32)
    ones16 = jnp.ones((16,), jnp.int32)
    qw16 = jnp.full((16,), QW, jnp.int32)
    dump16 = qw16 + iota16

    for t in range(_NSETS):
        bs, bd, ns = bs_h[t], bd_h[t], ns_h[t]

        def chunk(m, carry, t=t, bs=bs, bd=bd):
            pltpu.sync_copy(src_h[t].at[wid, pl.ds(m * KC, KC)], sidxc)
            pltpu.sync_copy(dst_h[t].at[wid, pl.ds(m * KC, KC)], didxc)

            def row(j, c, bs=bs, bd=bd):
                cur = list(c[:NQ])
                fill = list(c[NQ:])
                for l in range(B // 16):
                    sv = sidxc[j, pl.ds(16 * l, 16)]
                    dv = didxc[j, pl.ds(16 * l, 16)]
                    qv = (jnp.where(dv >= qw16, ones16, zeros16)
                          + jnp.where(dv >= qw16 + qw16, ones16, zeros16)
                          + jnp.where(dv >= qw16 + qw16 + qw16, ones16,
                                      zeros16))
                    rel = dv - qv * qw16
                    for qq in range(NQ):
                        mask = qv == jnp.full((16,), qq, jnp.int32)
                        inc = plsc.cumsum(
                            jnp.where(mask, ones16, zeros16))
                        cnt = jnp.max(inc)
                        pos = inc + jnp.full((16,), fill[qq] - 1,
                                             jnp.int32)
                        plsc.store_scatter(stg_s[qq], [pos], sv, mask=mask)
                        plsc.store_scatter(stg_d[qq], [pos], rel, mask=mask)
                        nf = fill[qq] + cnt
                        do_flush = nf >= CHE

                        @pl.when(do_flush)
                        def _(qq=qq, bs=bs, bd=bd, cur=cur):
                            off = cur[qq] * CHE
                            pltpu.sync_copy(
                                stg_s[qq].at[pl.ds(0, CHE)],
                                bs.at[wid, qq, pl.ds(off, CHE)])
                            pltpu.sync_copy(
                                stg_d[qq].at[pl.ds(0, CHE)],
                                bd.at[wid, qq, pl.ds(off, CHE)])
                            rs = stg_s[qq][pl.ds(CHE, 16)]
                            rd = stg_d[qq][pl.ds(CHE, 16)]
                            stg_s[qq][pl.ds(0, 16)] = rs
                            stg_d[qq][pl.ds(0, 16)] = rd

                        flushed = do_flush.astype(jnp.int32)
                        cur[qq] = cur[qq] + flushed
                        fill[qq] = nf - CHE * flushed
                return tuple(cur) + tuple(fill)

            return lax.fori_loop(0, KC, row, carry)

        z = jnp.int32(0)
        carry = lax.fori_loop(0, K // KC, chunk,
                              (z, z, z, z, z, z, z, z))
        nsum = zeros16
        for qq in range(NQ):
            cur, fill = carry[qq], carry[NQ + qq]
            for k in range(CHE // 16):
                stg_s[qq][pl.ds(fill + 16 * k, 16)] = zeros16
                stg_d[qq][pl.ds(fill + 16 * k, 16)] = dump16

            @pl.when(fill > 0)
            def _(qq=qq, cur=cur, bs=bs, bd=bd):
                off = cur * CHE
                pltpu.sync_copy(stg_s[qq].at[pl.ds(0, CHE)],
                                bs.at[wid, qq, pl.ds(off, CHE)])
                pltpu.sync_copy(stg_d[qq].at[pl.ds(0, CHE)],
                                bd.at[wid, qq, pl.ds(off, CHE)])

            steps = cur + (fill > 0).astype(jnp.int32)
            nsum = nsum + jnp.where(iota16 == jnp.full((16,), qq,
                                                       jnp.int32),
                                    jnp.full((16,), steps, jnp.int32),
                                    zeros16)
        nsv[...] = nsum
        pltpu.sync_copy(nsv, ns.at[wid])


def _make_sc_accum(num_types):
    nt = num_types

    @functools.partial(
        pl.kernel,
        out_type=jax.ShapeDtypeStruct((2, NQ, ACCR, H), jnp.float32),
        mesh=_MESH,
        compiler_params=_UNTILED,
        scratch_types=[
            pltpu.VMEM_SHARED((ACCR, H), jnp.float32),
            pltpu.VMEM((CHE,), jnp.int32),
            pltpu.VMEM((CHE,), jnp.int32),
            pltpu.VMEM((B, H), jnp.float32),
            pltpu.VMEM((B, H), jnp.float32),
            pltpu.VMEM((ZROWS, H), jnp.float32),
            pltpu.VMEM((16,), jnp.int32),
            pltpu.SemaphoreType.DMA,
            pltpu.SemaphoreType.DMA,
        ],
    )
    def body(*refs):
        g_refs = refs[:nt]
        bs_h = refs[nt:2 * nt]
        bd_h = refs[2 * nt:3 * nt]
        ns_h = refs[3 * nt:4 * nt]
        p_ref = refs[4 * nt]
        acc, sidxc, didxc, rows0, rows1, zbuf, nsv, sem0, sem1 = \
            refs[4 * nt + 1:]

        ci = lax.axis_index("c")
        si = lax.axis_index("s")
        wid = si * 2 + ci
        iota16 = lax.iota(jnp.int32, 16)

        z16 = jnp.zeros((16,), jnp.float32)

        def zfill(r, carry):
            for qq in range(H // 16):
                zbuf[r, pl.ds(qq * 16, 16)] = z16
            return carry

        lax.fori_loop(0, ZROWS, zfill, 0)

        for q in range(NQ):
            def zero(z, carry):
                pltpu.sync_copy(zbuf, acc.at[pl.ds(si * RPSQ + z * ZROWS,
                                                   ZROWS)])
                return carry

            lax.fori_loop(0, NZ, zero, 0)
            plsc.subcore_barrier()
            for t in range(nt):
                g = g_refs[t]
                pltpu.sync_copy(ns_h[t].at[wid], nsv)
                steps = jnp.max(jnp.where(
                    iota16 == jnp.full((16,), q, jnp.int32), nsv[...],
                    jnp.zeros((16,), jnp.int32)))

                def chunk(cc, carry, t=t, q=q, g=g):
                    pltpu.sync_copy(bs_h[t].at[wid, q, pl.ds(cc * CHE, CHE)],
                                    sidxc)
                    pltpu.sync_copy(bd_h[t].at[wid, q, pl.ds(cc * CHE, CHE)],
                                    didxc)
                    pltpu.async_copy(g.at[sidxc.at[pl.ds(0, B)]], rows0, sem0)
                    for r in range(RPC):
                        cur, csem = ((rows0, sem0) if r % 2 == 0
                                     else (rows1, sem1))
                        nxt, xsem = ((rows1, sem1) if r % 2 == 0
                                     else (rows0, sem0))
                        if r + 1 < RPC:
                            pltpu.async_copy(
                                g.at[sidxc.at[pl.ds((r + 1) * B, B)]],
                                nxt, xsem)
                        pltpu.make_async_copy(
                            g.at[sidxc.at[pl.ds(r * B, B)]], cur, csem).wait()
                        pltpu.sync_copy(cur,
                                        acc.at[didxc.at[pl.ds(r * B, B)]],
                                        add=True)
                    return carry

                lax.fori_loop(0, steps, chunk, 0)
            plsc.subcore_barrier()
            pltpu.sync_copy(acc.at[pl.ds(si * RPSQ, RPSQ)],
                            p_ref.at[ci, q, pl.ds(si * RPSQ, RPSQ)])
            plsc.subcore_barrier()

    return body


_sc_accum_1 = _make_sc_accum(1)
_sc_accum_3 = _make_sc_accum(3)


def kernel(x_base, x_joint, x_foot, edge_index_base_gt_base,
           edge_index_base_gs_base, edge_index_base_to_joint,
           edge_index_joint_to_joint, edge_index_joint_to_foot,
           edge_index_foot_to_joint, params):
    sets = (("b2j", edge_index_base_to_joint),
            ("j2j", edge_index_joint_to_joint),
            ("f2j", edge_index_foot_to_joint),
            ("j2f", edge_index_joint_to_foot))
    srcs = [_pad_src(ei[0]) for _, ei in sets]
    dsts = [_pad_dst(ei[1]) for _, ei in sets]
    bkt = _sc_bucket(*srcs, *dsts)
    buckets = {name: (bkt[i], bkt[_NSETS + i], bkt[2 * _NSETS + i])
               for i, (name, _) in enumerate(sets)}

    h0 = {
        "base": _encode(x_base, params["enc_base_W"], params["enc_base_b"]),
        "joint": _encode(x_joint, params["enc_joint_W"], params["enc_joint_b"]),
        "foot": _encode(x_foot, params["enc_foot_W"], params["enc_foot_b"]),
    }

    p0 = {e: params["l0_%s" % n] for e, n in (
        ("b2j", "base_to_joint"), ("j2j", "joint_to_joint"),
        ("f2j", "foot_to_joint"), ("j2f", "joint_to_foot"))}
    p1_j2f = params["l1_joint_to_foot"]

    g_b2j = _pretransform(h0["base"], p0["b2j"]["Wrel"])
    g_j2j = _pretransform(h0["joint"], p0["j2j"]["Wrel"])
    g_f2j = _pretransform(h0["foot"], p0["f2j"]["Wrel"])
    part_j = _sc_accum_3(g_b2j, g_j2j, g_f2j,
                         buckets["b2j"][0], buckets["j2j"][0],
                         buckets["f2j"][0],
                         buckets["b2j"][1], buckets["j2j"][1],
                         buckets["f2j"][1],
                         buckets["b2j"][2], buckets["j2j"][2],
                         buckets["f2j"][2])
    g_j2f = _pretransform(h0["joint"], p0["j2f"]["Wrel"])
    part_f = _sc_accum_1(g_j2f, *buckets["j2f"])

    w_root_j = p0["b2j"]["Wroot"] + p0["j2j"]["Wroot"] + p0["f2j"]["Wroot"]
    b_rel_j = p0["b2j"]["brel"] + p0["j2j"]["brel"] + p0["f2j"]["brel"]
    (h1_joint,) = _assemble(part_j, h0["joint"], w_root_j, b_rel_j)
    (h1_foot,) = _assemble(part_f, h0["foot"], p0["j2f"]["Wroot"],
                           p0["j2f"]["brel"])

    g1_j2f = _pretransform(h1_joint, p1_j2f["Wrel"])
    part_f1 = _sc_accum_1(g1_j2f, *buckets["j2f"])
    _, y = _assemble(part_f1, h1_foot, p1_j2f["Wroot"], p1_j2f["brel"],
                     dec_w=params["dec_W"], dec_b=params["dec_b"])
    return y
